# baseline (speedup 1.0000x reference)
import functools

import jax
import jax.numpy as jnp
import numpy as np
from jax.experimental import pallas as pl
from jax.experimental.pallas import tpu as pltpu

_AGENT_DIMS = [256, 384, 512, 512]
_WSI = 512
_T = 16
_N_CLASSES = 4
_E = 4
_CAP = 4
_AUX_W = 0.01
_P = 2048
_PH = _P // 2

_HIGH = jax.lax.Precision.HIGHEST
_DEF = jax.lax.Precision.DEFAULT


def _dot_t(a, b, precision=_HIGH):
    return jax.lax.dot_general(a, b, (((1,), (1,)), ((), ())),
                               precision=precision,
                               preferred_element_type=jnp.float32)


def _dot(a, b, precision=_HIGH):
    return jax.lax.dot_general(a, b, (((1,), (0,)), ((), ())),
                               precision=precision,
                               preferred_element_type=jnp.float32)


def _ln_rows(x, g, b, eps=1e-5):
    m = jnp.mean(x, axis=1, keepdims=True)
    v = jnp.mean((x - m) ** 2, axis=1, keepdims=True)
    return (x - m) / jnp.sqrt(v + eps) * g + b


def _stage_a_body(*refs):
    ins = refs[:32]
    outs = refs[32:36]
    qm_s, kv_s, os_s, ss_s = refs[36:40]
    for li, d in enumerate(_AGENT_DIMS):
        (sf_ref, tok_ref, l1g, l1b, l2g, l2b, inw_ref, inb_ref) = \
            ins[8 * li:8 * li + 8]
        out_ref = outs[li]

        sfn = _ln_rows(sf_ref[...], l1g[...], l1b[...])
        tokn = _ln_rows(tok_ref[...], l2g[...], l2b[...])
        inb = inb_ref[...]
        qp = _dot_t(tokn, inw_ref[0:d, :]) + inb[:, 0:d]
        kp = _dot_t(sfn, inw_ref[d:2 * d, :]) + inb[:, d:2 * d]
        vp = _dot_t(sfn, inw_ref[2 * d:3 * d, :]) + inb[:, 2 * d:3 * d]

        cmax = jnp.max(kp, axis=0, keepdims=True)
        cmin = jnp.min(kp, axis=0, keepdims=True)
        m16 = jnp.maximum(qp * cmax, qp * cmin)

        kv_s[0:_PH, 0:d] = kp
        kv_s[_PH:2 * _PH, 0:d] = vp
        qm_s[0:_T, 0:d] = qp
        qm_s[_T:2 * _T, 0:d] = m16

        ones = jnp.ones((1, _PH), jnp.float32)

        def tok_body(t, carry):
            q = qm_s[pl.ds(t, 1), 0:d]
            m = qm_s[pl.ds(t + _T, 1), 0:d]
            kpv = kv_s[0:_PH, 0:d]
            vpv = kv_s[_PH:2 * _PH, 0:d]
            e = jnp.exp(kpv * q - m)
            ss_s[pl.ds(t, 1), 0:d] = _dot(ones, e)
            os_s[pl.ds(t, 1), 0:d] = _dot(ones, e * vpv)
            return carry

        jax.lax.fori_loop(0, _T, tok_body, 0)
        out_ref[0, 0:_T, :] = os_s[0:_T, 0:d]
        out_ref[0, _T:2 * _T, :] = ss_s[0:_T, 0:d]
        out_ref[0, 2 * _T:3 * _T, :] = m16


def _stage_a(share_feature, layers):
    in_specs = []
    args = []
    off = 0
    for i, d in enumerate(_AGENT_DIMS):
        lp = layers[i]
        sf = share_feature[:, off:off + d]
        off += d
        args += [sf, lp['tok'],
                 lp['ln1_g'].reshape(1, d), lp['ln1_b'].reshape(1, d),
                 lp['ln2_g'].reshape(1, d), lp['ln2_b'].reshape(1, d),
                 lp['in_w'], lp['in_b'].reshape(1, 3 * d)]
        in_specs += [
            pl.BlockSpec((_PH, d), lambda c: (c, 0)),
            pl.BlockSpec((_T, d), lambda c: (0, 0)),
            pl.BlockSpec((1, d), lambda c: (0, 0)),
            pl.BlockSpec((1, d), lambda c: (0, 0)),
            pl.BlockSpec((1, d), lambda c: (0, 0)),
            pl.BlockSpec((1, d), lambda c: (0, 0)),
            pl.BlockSpec((3 * d, d), lambda c: (0, 0)),
            pl.BlockSpec((1, 3 * d), lambda c: (0, 0)),
        ]
    out_shapes = [jax.ShapeDtypeStruct((2, 3 * _T, d), jnp.float32)
                  for d in _AGENT_DIMS]
    out_specs = [pl.BlockSpec((1, 3 * _T, d), lambda c: (c, 0, 0))
                 for d in _AGENT_DIMS]
    return pl.pallas_call(
        _stage_a_body,
        grid=(2,),
        in_specs=in_specs,
        out_specs=out_specs,
        out_shape=out_shapes,
        scratch_shapes=[
            pltpu.VMEM((2 * _T, _WSI), jnp.float32),
            pltpu.VMEM((2 * _PH, _WSI), jnp.float32),
            pltpu.VMEM((_T, _WSI), jnp.float32),
            pltpu.VMEM((_T, _WSI), jnp.float32),
        ],
        compiler_params=pltpu.CompilerParams(
            dimension_semantics=("parallel",)),
    )(*args)


def _stage_b_body(*refs):
    (p0, p1, p2, p3,
     ow0, ob0, mw0, mb0, ow1, ob1, mw1, mb1,
     ow2, ob2, mw2, mb2, ow3, ob3, mw3, mb3,
     emb_ref, ls_ref, w1_ref, b1_ref, w2_ref, b2_ref,
     bl1g, bl1b, bl2g, bl2b, binw, binb, boutw, boutb,
     bfcw, bfcb, bpjw, bpjb, btw, btb, clw_ref, clb_ref,
     f1_out, aux_out, lg_out, yh_out) = refs

    parts = (p0, p1, p2, p3)
    proj = ((ow0, ob0, mw0, mb0), (ow1, ob1, mw1, mb1),
            (ow2, ob2, mw2, mb2), (ow3, ob3, mw3, mb3))

    xs = []
    for i, d in enumerate(_AGENT_DIMS):
        P = parts[i][...]
        o0, s0, m0 = P[0, 0:_T, :], P[0, _T:2 * _T, :], P[0, 2 * _T:, :]
        o1, s1, m1 = P[1, 0:_T, :], P[1, _T:2 * _T, :], P[1, 2 * _T:, :]
        mm = jnp.maximum(m0, m1)
        w0 = jnp.exp(m0 - mm)
        w1 = jnp.exp(m1 - mm)
        att = (o0 * w0 + o1 * w1) / (s0 * w0 + s1 * w1)
        ow, ob, mw, mb = proj[i]
        val = _dot_t(att, ow[...]) + ob[...]
        xs.append(_dot_t(val, mw[...]) + mb[...])
    X = jnp.concatenate(xs, axis=0)

    nrm = jnp.sqrt(jnp.sum(X * X, axis=1, keepdims=True))
    xn = X / (nrm + 1e-6)
    emb = emb_ref[...]
    enrm = jnp.sqrt(jnp.sum(emb * emb, axis=1, keepdims=True))
    en = emb / (enrm + 1e-6)
    scale = jnp.minimum(jnp.exp(ls_ref[...]), 100.0)
    logits = _dot_t(xn, en)[:, 0:_E] * scale

    lmax = jnp.max(logits, axis=1, keepdims=True)
    eg = jnp.exp(logits - lmax)
    gates = eg / jnp.sum(eg, axis=1, keepdims=True)
    gate_val = jnp.max(gates, axis=1, keepdims=True)
    lane4 = jax.lax.broadcasted_iota(jnp.int32, (4 * _T, _E), 1).astype(
        jnp.float32)
    is_max = (gates == gate_val)
    idxf = jnp.min(jnp.where(is_max, lane4, float(_E)), axis=1, keepdims=True)
    onehot = jnp.where(lane4 == idxf, 1.0, 0.0)

    sub16 = jax.lax.broadcasted_iota(jnp.int32, (_T, _T), 0)
    lane16 = jax.lax.broadcasted_iota(jnp.int32, (_T, _T), 1)

    aux_total = jnp.zeros((1, 1), jnp.float32)
    keepw = []
    for i in range(_E):
        g = gate_val[16 * i:16 * i + 16, :]
        oh = onehot[16 * i:16 * i + 16, :]
        gs = gates[16 * i:16 * i + 16, :]
        me = jnp.sum(gs, axis=0, keepdims=True) / float(_T)
        ce = jnp.sum(oh, axis=0, keepdims=True) / float(_T)
        aux_total = aux_total + jnp.sum(
            me * ce, axis=1, keepdims=True) * float(_E) * _AUX_W
        same = _dot_t(oh, oh)
        g_row = jnp.transpose(g, (1, 0))
        before = jnp.where(
            (lane16 != sub16)
            & ((g_row > g) | ((g_row == g) & (lane16 < sub16))), 1.0, 0.0)
        rank = jnp.sum(same * before, axis=1, keepdims=True)
        keepw.append(g * jnp.where(rank < float(_CAP), 1.0, 0.0))
    wgt = jnp.concatenate(keepw, axis=0)

    Xb = X.astype(jnp.bfloat16)
    mo = jnp.zeros((4 * _T, _WSI), jnp.float32)
    for e in range(_E):
        h = _dot(Xb, w1_ref[e, :, :], precision=_DEF)
        h = h + b1_ref[e:e + 1, :]
        h = 0.5 * h * (1.0 + jax.lax.erf(h * np.float32(1.0 / np.sqrt(2.0))))
        y = _dot(h.astype(jnp.bfloat16), w2_ref[e, :, :], precision=_DEF)
        y = y + b2_ref[e:e + 1, :]
        mo = mo + y * jnp.where(idxf == float(e), wgt, 0.0)

    cur = jnp.zeros((_T, _WSI), jnp.float32)
    for i in range(_E):
        cur = cur + mo[16 * i:16 * i + 16, :]
        f1_out[i, :, :] = cur

    sub8 = jax.lax.broadcasted_iota(jnp.int32, (8, 128), 0)
    lane8 = jax.lax.broadcasted_iota(jnp.int32, (8, 128), 1)
    aux_out[...] = jnp.where((sub8 == 0) & (lane8 == 0),
                             jnp.broadcast_to(aux_total, (8, 128)), 0.0)

    cur = cur * np.float32(1.0 / _E)
    xnl = _ln_rows(cur, bl1g[...], bl1b[...])
    binbv = binb[...]
    qp = _dot_t(xnl, binw[0:_WSI, :]) + binbv[:, 0:_WSI]
    kp = _dot_t(xnl, binw[_WSI:2 * _WSI, :]) + binbv[:, _WSI:2 * _WSI]
    vp = _dot_t(xnl, binw[2 * _WSI:3 * _WSI, :]) + binbv[:, 2 * _WSI:3 * _WSI]
    hd = 64
    heads = []
    for h8 in range(_WSI // hd):
        qh = qp[:, hd * h8:hd * (h8 + 1)]
        kh = kp[:, hd * h8:hd * (h8 + 1)]
        vh = vp[:, hd * h8:hd * (h8 + 1)]
        sc = _dot_t(qh, kh) * np.float32(1.0 / np.sqrt(hd))
        sc = sc - jnp.max(sc, axis=1, keepdims=True)
        es = jnp.exp(sc)
        att = es / jnp.sum(es, axis=1, keepdims=True)
        heads.append(_dot(att, vh))
    o = jnp.concatenate(heads, axis=1)
    xnew = cur + _dot_t(o, boutw[...]) + boutb[...]

    x2 = _ln_rows(xnew, bl2g[...], bl2b[...])
    hmid = _dot_t(x2, bfcw[...]) + bfcb[...]
    hmid = hmid * (1.0 / (1.0 + jnp.exp(-1.702 * hmid)))
    gated = xnew + _dot_t(hmid, bpjw[...]) + bpjb[...]

    hb = _dot_t(gated, btw[...]) + btb[...]
    clb = clb_ref[...]
    ls = []
    for c in range(_N_CLASSES):
        ls.append(jnp.sum(hb * clw_ref[c, :, :]) + clb[0, c])
    best = jnp.maximum(jnp.maximum(ls[0], ls[1]), jnp.maximum(ls[2], ls[3]))
    yh = jnp.where(
        ls[0] == best, 0,
        jnp.where(ls[1] == best, 1, jnp.where(ls[2] == best, 2, 3)))

    arr = jnp.zeros((8, 128), jnp.float32)
    for c in range(_N_CLASSES):
        arr = jnp.where((sub8 == 0) & (lane8 == c), ls[c], arr)
        hz = 1.0 / (1.0 + jnp.exp(-ls[c]))
        arr = jnp.where((sub8 == 1) & (lane8 == c), hz, arr)
    lg_out[...] = arr
    yh_out[...] = jnp.where((sub8 == 0) & (lane8 == 0), yh, 0)


def _stage_b(partials, layers, mp, bp, op):
    emb_pad = jnp.zeros((8, _WSI), jnp.float32).at[0:_E].set(mp['emb'])
    args = list(partials)
    for i, d in enumerate(_AGENT_DIMS):
        lp = layers[i]
        args += [lp['out_w'], lp['out_b'].reshape(1, d),
                 lp['mlp_w'], lp['mlp_b'].reshape(1, _WSI)]
    args += [emb_pad, mp['log_scale'].reshape(1, 1),
             mp['w1'].astype(jnp.bfloat16), mp['b1'],
             mp['w2'].astype(jnp.bfloat16), mp['b2'],
             bp['ln1_g'].reshape(1, _WSI), bp['ln1_b'].reshape(1, _WSI),
             bp['ln2_g'].reshape(1, _WSI), bp['ln2_b'].reshape(1, _WSI),
             bp['in_w'], bp['in_b'].reshape(1, 3 * _WSI),
             bp['out_w'], bp['out_b'].reshape(1, _WSI),
             bp['fc_w'], bp['fc_b'].reshape(1, 4 * _WSI),
             bp['proj_w'], bp['proj_b'].reshape(1, _WSI),
             op['butter_w'], op['butter_b'].reshape(1, 256),
             op['cls_w'].reshape(_N_CLASSES, _T, 256),
             op['cls_b'].reshape(1, _N_CLASSES)]
    return pl.pallas_call(
        _stage_b_body,
        out_shape=[
            jax.ShapeDtypeStruct((_E, _T, _WSI), jnp.float32),
            jax.ShapeDtypeStruct((8, 128), jnp.float32),
            jax.ShapeDtypeStruct((8, 128), jnp.float32),
            jax.ShapeDtypeStruct((8, 128), jnp.int32),
        ],
    )(*args)


def kernel(share_feature, params):
    partials = _stage_a(share_feature, params['layers'])
    feature1, aux, lg, yh = _stage_b(partials, params['layers'],
                                     params['moe'], params['blk'],
                                     params['out'])
    moe_aux = aux[0, 0:1]
    logits = lg[0:1, 0:_N_CLASSES]
    hazards = lg[1:2, 0:_N_CLASSES]
    Y_hat = yh[0:1, 0:1]
    feature2 = jnp.zeros((_E, _T, _WSI), dtype=share_feature.dtype)
    feature2_pre = jnp.zeros((_E, 1, _N_CLASSES), dtype=share_feature.dtype)
    return (logits, hazards, Y_hat, moe_aux, feature1, feature2, feature2_pre)
